# Initial kernel scaffold; baseline (speedup 1.0000x reference)
import functools

import jax
import jax.numpy as jnp
from jax import lax
from jax.experimental import pallas as pl
from jax.experimental.pallas import tpu as pltpu
from jax.experimental.pallas import tpu_sc as plsc

_NUM_TASKS = 12
_TOKEN_DIM = 128
_BATCH = 16384
_NC = 2
_NS = 16
_NW = _NC * _NS
_BPW = _BATCH // _NW
_L = 16
_NCHUNK = _BPW // 128

_mesh = plsc.VectorSubcoreMesh(core_axis_name="c", subcore_axis_name="s")


@functools.partial(
    pl.kernel,
    out_type=jax.ShapeDtypeStruct((_BATCH, _TOKEN_DIM), jnp.float32),
    mesh=_mesh,
    scratch_types=[
        pltpu.VMEM((_BPW,), jnp.int32),
        pltpu.VMEM((_BPW,), jnp.int32),
        pltpu.VMEM((10, 3), jnp.int32),
        pltpu.VMEM((_NCHUNK, 128), jnp.int32),
        pltpu.VMEM((_BPW, _TOKEN_DIM), jnp.float32),
        pltpu.SemaphoreType.DMA,
        pltpu.SemaphoreType.DMA,
    ],
)
def _task_encoder_sc(tm_hbm, pn_hbm, emb_hbm, lut_hbm, out_hbm,
                     tm_v, pn_v, lut_v, idx_v, rows_v, gsem, osem):
    wid = lax.axis_index("s") * _NC + lax.axis_index("c")
    base = wid * _BPW

    pltpu.sync_copy(tm_hbm.at[pl.ds(base, _BPW)], tm_v)
    pltpu.sync_copy(pn_hbm.at[pl.ds(base, _BPW)], pn_v)
    pltpu.sync_copy(lut_hbm, lut_v)

    for i in range(_BPW // _L):
        tm = tm_v[pl.ds(i * _L, _L)]
        pn = pn_v[pl.ds(i * _L, _L)]
        idx = plsc.load_gather(lut_v, [tm, pn])
        idx_v[i // 8, pl.ds((i % 8) * _L, _L)] = idx

    copies = []
    for j in range(_NCHUNK):
        copies.append(
            pltpu.async_copy(
                emb_hbm.at[idx_v.at[j]],
                rows_v.at[pl.ds(j * 128, 128)],
                gsem,
            )
        )
    out_copies = []
    for j in range(_NCHUNK):
        copies[j].wait()
        out_copies.append(
            pltpu.async_copy(
                rows_v.at[pl.ds(j * 128, 128)],
                out_hbm.at[pl.ds(base + j * 128, 128)],
                osem,
            )
        )
    for c in out_copies:
        c.wait()


def kernel(target_module, port_name, embedding, lookup):
    out = _task_encoder_sc(target_module, port_name, embedding, lookup)
    return out[:, None, :]


# speedup vs baseline: 2.3346x; 2.3346x over previous
import functools

import jax
import jax.numpy as jnp
from jax import lax
from jax.experimental import pallas as pl
from jax.experimental.pallas import tpu as pltpu
from jax.experimental.pallas import tpu_sc as plsc

_NUM_TASKS = 12
_TOKEN_DIM = 128
_BATCH = 16384
_NC = 2
_NS = 16
_NW = _NC * _NS
_BPW = _BATCH // _NW
_L = 16
_NCHUNK = _BPW // 128

_mesh = plsc.VectorSubcoreMesh(core_axis_name="c", subcore_axis_name="s")


@functools.partial(
    pl.kernel,
    out_type=jax.ShapeDtypeStruct((_BATCH, _TOKEN_DIM), jnp.float32),
    mesh=_mesh,
    compiler_params=pltpu.CompilerParams(needs_layout_passes=False),
    scratch_types=[
        pltpu.VMEM((_BPW,), jnp.int32),
        pltpu.VMEM((_BPW,), jnp.int32),
        pltpu.VMEM((10, 3), jnp.int32),
        pltpu.VMEM((_NCHUNK, 128), jnp.int32),
        pltpu.VMEM((_BPW, _TOKEN_DIM), jnp.float32),
        pltpu.SemaphoreType.DMA,
        pltpu.SemaphoreType.DMA,
    ],
)
def _task_encoder_sc(tm_hbm, pn_hbm, emb_hbm, lut_hbm, out_hbm,
                     tm_v, pn_v, lut_v, idx_v, rows_v, gsem, osem):
    wid = lax.axis_index("s") * _NC + lax.axis_index("c")
    base = wid * _BPW

    pltpu.sync_copy(tm_hbm.at[pl.ds(base, _BPW)], tm_v)
    pltpu.sync_copy(pn_hbm.at[pl.ds(base, _BPW)], pn_v)
    pltpu.sync_copy(lut_hbm, lut_v)

    for i in range(_BPW // _L):
        tm = tm_v[pl.ds(i * _L, _L)]
        pn = pn_v[pl.ds(i * _L, _L)]
        idx = plsc.load_gather(lut_v, [tm, pn])
        idx_v[i // 8, pl.ds((i % 8) * _L, _L)] = idx

    copies = []
    for j in range(_NCHUNK):
        copies.append(
            pltpu.async_copy(
                emb_hbm.at[idx_v.at[j]],
                rows_v.at[pl.ds(j * 128, 128)],
                gsem,
            )
        )
    out_copies = []
    for j in range(_NCHUNK):
        copies[j].wait()
        out_copies.append(
            pltpu.async_copy(
                rows_v.at[pl.ds(j * 128, 128)],
                out_hbm.at[pl.ds(base + j * 128, 128)],
                osem,
            )
        )
    for c in out_copies:
        c.wait()


def kernel(target_module, port_name, embedding, lookup):
    out = _task_encoder_sc(target_module, port_name, embedding, lookup)
    return out[:, None, :]


# speedup vs baseline: 4.9631x; 2.1259x over previous
import functools

import jax
import jax.numpy as jnp
from jax import lax
from jax.experimental import pallas as pl
from jax.experimental.pallas import tpu as pltpu
from jax.experimental.pallas import tpu_sc as plsc

_NUM_TASKS = 12
_TOKEN_DIM = 128
_BATCH = 16384
_NC = 2
_NS = 16
_NW = _NC * _NS
_BPW = _BATCH // _NW
_L = 16
_GROUP = 128
_NGROUP = _BPW // _GROUP

_mesh = plsc.VectorSubcoreMesh(core_axis_name="c", subcore_axis_name="s")


@functools.partial(
    pl.kernel,
    out_type=jax.ShapeDtypeStruct((_BATCH * _TOKEN_DIM,), jnp.float32),
    mesh=_mesh,
    compiler_params=pltpu.CompilerParams(needs_layout_passes=False),
    scratch_types=[
        pltpu.VMEM((_BPW,), jnp.int32),
        pltpu.VMEM((_BPW,), jnp.int32),
        pltpu.VMEM((10, 3), jnp.int32),
        pltpu.VMEM((_BPW,), jnp.int32),
        pltpu.VMEM((_NUM_TASKS * _TOKEN_DIM,), jnp.float32),
        pltpu.VMEM((2, _GROUP * _TOKEN_DIM), jnp.float32),
        pltpu.SemaphoreType.DMA,
        pltpu.SemaphoreType.DMA,
    ],
)
def _task_encoder_sc(tm_hbm, pn_hbm, emb_hbm, lut_hbm, out_hbm,
                     tm_v, pn_v, lut_v, idx_v, emb_v, buf_v, sem0, sem1):
    wid = lax.axis_index("s") * _NC + lax.axis_index("c")
    base = wid * _BPW

    pltpu.sync_copy(tm_hbm.at[pl.ds(base, _BPW)], tm_v)
    pltpu.sync_copy(pn_hbm.at[pl.ds(base, _BPW)], pn_v)
    pltpu.sync_copy(lut_hbm, lut_v)
    pltpu.sync_copy(emb_hbm, emb_v)

    for i in range(_BPW // _L):
        tm = tm_v[pl.ds(i * _L, _L)]
        pn = pn_v[pl.ds(i * _L, _L)]
        idx_v[pl.ds(i * _L, _L)] = plsc.load_gather(lut_v, [tm, pn])

    sems = (sem0, sem1)
    copies = [None, None]
    for g in range(_NGROUP):
        p = g % 2
        if copies[p] is not None:
            copies[p].wait()

        @pl.loop(0, _GROUP // _L)
        def _expand(i, g=g, p=p):
            offs = idx_v[pl.ds(g * _GROUP + i * _L, _L)] * _TOKEN_DIM
            for k in range(_L):
                off = offs[k]
                dst = (i * _L + k) * _TOKEN_DIM
                for j in range(_TOKEN_DIM // _L):
                    buf_v[p, pl.ds(dst + j * _L, _L)] = emb_v[pl.ds(off + j * _L, _L)]

        copies[p] = pltpu.async_copy(
            buf_v.at[p],
            out_hbm.at[pl.ds((base + g * _GROUP) * _TOKEN_DIM, _GROUP * _TOKEN_DIM)],
            sems[p],
        )
    copies[0].wait()
    copies[1].wait()


def kernel(target_module, port_name, embedding, lookup):
    out = _task_encoder_sc(
        target_module, port_name, embedding.reshape(-1), lookup
    )
    return out.reshape(_BATCH, 1, _TOKEN_DIM)


# speedup vs baseline: 5.7664x; 1.1619x over previous
import functools

import jax
import jax.numpy as jnp
from jax import lax
from jax.experimental import pallas as pl
from jax.experimental.pallas import tpu as pltpu
from jax.experimental.pallas import tpu_sc as plsc

_NUM_TASKS = 12
_TOKEN_DIM = 128
_BATCH = 16384
_NC = 2
_NS = 16
_NW = _NC * _NS
_BPW = _BATCH // _NW
_L = 16
_GROUP = 128
_NGROUP = _BPW // _GROUP

_mesh = plsc.VectorSubcoreMesh(core_axis_name="c", subcore_axis_name="s")


@functools.partial(
    pl.kernel,
    out_type=jax.ShapeDtypeStruct((_BATCH * _TOKEN_DIM,), jnp.float32),
    mesh=_mesh,
    compiler_params=pltpu.CompilerParams(needs_layout_passes=False),
    scratch_types=[
        pltpu.VMEM((_BPW,), jnp.int32),
        pltpu.VMEM((_BPW,), jnp.int32),
        pltpu.VMEM((10, 3), jnp.int32),
        pltpu.VMEM((_BPW,), jnp.int32),
        pltpu.VMEM((_NUM_TASKS * _TOKEN_DIM,), jnp.float32),
        pltpu.VMEM((2, _GROUP * _TOKEN_DIM), jnp.float32),
        pltpu.SemaphoreType.DMA,
        pltpu.SemaphoreType.DMA,
    ],
)
def _task_encoder_sc(tm_hbm, pn_hbm, emb_hbm, lut_hbm, out_hbm,
                     tm_v, pn_v, lut_v, idx_v, emb_v, buf_v, sem0, sem1):
    wid = lax.axis_index("s") * _NC + lax.axis_index("c")
    base = wid * _BPW

    pltpu.sync_copy(tm_hbm.at[pl.ds(base, _BPW)], tm_v)
    pltpu.sync_copy(pn_hbm.at[pl.ds(base, _BPW)], pn_v)
    pltpu.sync_copy(lut_hbm, lut_v)
    pltpu.sync_copy(emb_hbm, emb_v)

    for i in range(_BPW // _L):
        tm = tm_v[pl.ds(i * _L, _L)]
        pn = pn_v[pl.ds(i * _L, _L)]
        idx_v[pl.ds(i * _L, _L)] = plsc.load_gather(lut_v, [tm, pn])

    sems = (sem0, sem1)
    copies = [None, None]
    for g in range(_NGROUP):
        p = g % 2
        if copies[p] is not None:
            copies[p].wait()

        @plsc.parallel_loop(0, _GROUP // _L, unroll=2)
        def _expand(i, g=g, p=p):
            offs = idx_v[pl.ds(g * _GROUP + i * _L, _L)] * _TOKEN_DIM
            for k in range(_L):
                off = offs[k]
                dst = (i * _L + k) * _TOKEN_DIM
                for j in range(_TOKEN_DIM // _L):
                    buf_v[p, pl.ds(dst + j * _L, _L)] = emb_v[pl.ds(off + j * _L, _L)]

        copies[p] = pltpu.async_copy(
            buf_v.at[p],
            out_hbm.at[pl.ds((base + g * _GROUP) * _TOKEN_DIM, _GROUP * _TOKEN_DIM)],
            sems[p],
        )
    copies[0].wait()
    copies[1].wait()


def kernel(target_module, port_name, embedding, lookup):
    out = _task_encoder_sc(
        target_module, port_name, embedding.reshape(-1), lookup
    )
    return out.reshape(_BATCH, 1, _TOKEN_DIM)
